# baseline (speedup 1.0000x reference)
import numpy as np

import jax
import jax.numpy as jnp
from jax.experimental import pallas as pl

L = 8192
N1, N2 = 64, 128
C = 2
J = 24
K = 50
RB = 32

_P = jax.lax.Precision.HIGHEST


def _f32(a):
    return jnp.asarray(a, jnp.float32)


_a1 = np.arange(N1)
_a2 = np.arange(N2)
_W1 = np.exp(-2j * np.pi * np.outer(_a1, _a1) / N1)
_W2 = np.exp(-2j * np.pi * np.outer(_a2, _a2) / N2)
_TW = np.exp(-2j * np.pi * np.outer(_a2, _a1) / L)


def _gmat_kernel(wt_ref, w1r_ref, w1i_ref, w2r_ref, w2i_ref, twr_ref, twi_ref,
                 gm_ref):
    w1r, w1i = w1r_ref[...], w1i_ref[...]
    w2r, w2i = w2r_ref[...], w2i_ref[...]
    twr, twi = twr_ref[...], twi_ref[...]

    xm = wt_ref[...].reshape(C * J, N1, N2)
    dn = (((1,), (0,)), ((), ()))
    yr = jax.lax.dot_general(xm, w1r, dn, precision=_P,
                             preferred_element_type=jnp.float32)
    yi = jax.lax.dot_general(xm, w1i, dn, precision=_P,
                             preferred_element_type=jnp.float32)
    zr = yr * twr[None] - yi * twi[None]
    zi = yr * twi[None] + yi * twr[None]
    fr = (jax.lax.dot_general(zr, w2r, dn, precision=_P,
                              preferred_element_type=jnp.float32)
          - jax.lax.dot_general(zi, w2i, dn, precision=_P,
                                preferred_element_type=jnp.float32))
    fi = (jax.lax.dot_general(zr, w2i, dn, precision=_P,
                              preferred_element_type=jnp.float32)
          + jax.lax.dot_general(zi, w2r, dn, precision=_P,
                                preferred_element_type=jnp.float32))
    fr = jnp.swapaxes(fr, 1, 2).reshape(C, J, L)
    fi = jnp.swapaxes(fi, 1, 2).reshape(C, J, L)
    scale = jnp.float32(1.0 / L)
    for c in range(C):
        gm_ref[c, 0, :, :] = jnp.swapaxes(fr[c], 0, 1) * scale
        gm_ref[c, 1, :, :] = jnp.swapaxes(fi[c], 0, 1) * scale


def _main_kernel(mag_ref, xv_ref, gm_ref, b_ref, out_ref):
    mag = mag_ref[...]
    bits = jax.lax.bitcast_convert_type(mag, jnp.int32)
    cur = jnp.zeros((RB, C, 1), jnp.int32)
    for bit in range(30, -1, -1):
        cand = cur | jnp.int32(1 << bit)
        cnt = jnp.sum((bits >= cand).astype(jnp.int32), axis=-1, keepdims=True)
        cur = jnp.where(cnt >= K, cand, cur)
    kth = jax.lax.bitcast_convert_type(cur, jnp.float32)
    mask = (mag >= kth).astype(jnp.float32)

    rs = xv_ref[...] * mask[:, :, None, :]
    a = jnp.concatenate(
        [rs[:, 0, 0], rs[:, 0, 1], rs[:, 1, 0], rs[:, 1, 1]], axis=-1)
    gm = gm_ref[...]
    dn = (((1,), (0,)), ((), ()))
    out = jax.lax.dot_general(a, gm, dn, precision=_P,
                              preferred_element_type=jnp.float32)
    out_ref[...] = out + b_ref[...][None, :]


def kernel(x01, W, b):
    B = x01.shape[0]
    x = 2.0 * x01.astype(jnp.float32) - 1.0
    X = jnp.fft.fft(x, axis=-1)
    mag = jnp.abs(X)
    xv = jnp.stack([jnp.real(X), jnp.imag(X)], axis=2)

    wt = W.astype(jnp.float32).reshape(C, L, J).transpose(0, 2, 1).reshape(C * J, L)
    gm = pl.pallas_call(
        _gmat_kernel,
        out_shape=jax.ShapeDtypeStruct((C, 2, L, J), jnp.float32),
    )(wt, _f32(_W1.real), _f32(_W1.imag), _f32(_W2.real), _f32(_W2.imag),
      _f32(_TW.real), _f32(_TW.imag))

    gmi = gm.reshape(2 * C * L, J)

    grid = (B // RB,)
    out = pl.pallas_call(
        _main_kernel,
        grid=grid,
        in_specs=[
            pl.BlockSpec((RB, C, L), lambda i: (i, 0, 0)),
            pl.BlockSpec((RB, C, 2, L), lambda i: (i, 0, 0, 0)),
            pl.BlockSpec((2 * C * L, J), lambda i: (0, 0)),
            pl.BlockSpec((J,), lambda i: (0,)),
        ],
        out_specs=pl.BlockSpec((RB, J), lambda i: (i, 0)),
        out_shape=jax.ShapeDtypeStruct((B, J), jnp.float32),
    )(mag, xv, gmi, b.astype(jnp.float32))
    return out


# speedup vs baseline: 1.0917x; 1.0917x over previous
import numpy as np

import jax
import jax.numpy as jnp
from jax.experimental import pallas as pl

L = 8192
N1, N2 = 64, 128
C = 2
J = 24
K = 50
RB = 32

_P = jax.lax.Precision.HIGHEST


def _f32(a):
    return jnp.asarray(a, jnp.float32)


_a1 = np.arange(N1)
_a2 = np.arange(N2)
_W1 = np.exp(-2j * np.pi * np.outer(_a1, _a1) / N1)
_W2 = np.exp(-2j * np.pi * np.outer(_a2, _a2) / N2)
_TW = np.exp(-2j * np.pi * np.outer(_a2, _a1) / L)


def _gmat_kernel(wt_ref, w1r_ref, w1i_ref, w2r_ref, w2i_ref, twr_ref, twi_ref,
                 gm_ref):
    w1r, w1i = w1r_ref[...], w1i_ref[...]
    w2r, w2i = w2r_ref[...], w2i_ref[...]
    twr, twi = twr_ref[...], twi_ref[...]

    xm = wt_ref[...].reshape(C * J, N1, N2)
    dn = (((1,), (0,)), ((), ()))
    yr = jax.lax.dot_general(xm, w1r, dn, precision=_P,
                             preferred_element_type=jnp.float32)
    yi = jax.lax.dot_general(xm, w1i, dn, precision=_P,
                             preferred_element_type=jnp.float32)
    zr = yr * twr[None] - yi * twi[None]
    zi = yr * twi[None] + yi * twr[None]
    fr = (jax.lax.dot_general(zr, w2r, dn, precision=_P,
                              preferred_element_type=jnp.float32)
          - jax.lax.dot_general(zi, w2i, dn, precision=_P,
                                preferred_element_type=jnp.float32))
    fi = (jax.lax.dot_general(zr, w2i, dn, precision=_P,
                              preferred_element_type=jnp.float32)
          + jax.lax.dot_general(zi, w2r, dn, precision=_P,
                                preferred_element_type=jnp.float32))
    fr = jnp.swapaxes(fr, 1, 2).reshape(C, J, L)
    fi = jnp.swapaxes(fi, 1, 2).reshape(C, J, L)
    scale = jnp.float32(1.0 / L)
    for c in range(C):
        gm_ref[c, 0, :, :] = jnp.swapaxes(fr[c], 0, 1) * scale
        gm_ref[c, 1, :, :] = jnp.swapaxes(fi[c], 0, 1) * scale


def _main_kernel(mag_ref, re_ref, im_ref, gm_ref, b_ref, out_ref):
    mag = mag_ref[...]
    bits = jax.lax.bitcast_convert_type(mag, jnp.int32)
    cur = jnp.zeros((RB, C, 1), jnp.int32)
    for bit in range(30, -1, -1):
        cand = cur | jnp.int32(1 << bit)
        cnt = jnp.sum((bits >= cand).astype(jnp.int32), axis=-1, keepdims=True)
        cur = jnp.where(cnt >= K, cand, cur)
    kth = jax.lax.bitcast_convert_type(cur, jnp.float32)
    mask = (mag >= kth).astype(jnp.float32)

    are = re_ref[...] * mask
    aim = im_ref[...] * mask
    a = jnp.concatenate(
        [are[:, 0], aim[:, 0], are[:, 1], aim[:, 1]], axis=-1)
    gm = gm_ref[...]
    dn = (((1,), (0,)), ((), ()))
    out = jax.lax.dot_general(a, gm, dn, precision=_P,
                              preferred_element_type=jnp.float32)
    out_ref[...] = out + b_ref[...][None, :]


def kernel(x01, W, b):
    B = x01.shape[0]
    x = 2.0 * x01.astype(jnp.float32) - 1.0
    X = jnp.fft.fft(x, axis=-1)
    mag = jnp.abs(X)
    re = jnp.real(X)
    im = jnp.imag(X)

    wt = W.astype(jnp.float32).reshape(C, L, J).transpose(0, 2, 1).reshape(C * J, L)
    gm = pl.pallas_call(
        _gmat_kernel,
        out_shape=jax.ShapeDtypeStruct((C, 2, L, J), jnp.float32),
    )(wt, _f32(_W1.real), _f32(_W1.imag), _f32(_W2.real), _f32(_W2.imag),
      _f32(_TW.real), _f32(_TW.imag))

    gmi = gm.reshape(2 * C * L, J)

    grid = (B // RB,)
    out = pl.pallas_call(
        _main_kernel,
        grid=grid,
        in_specs=[
            pl.BlockSpec((RB, C, L), lambda i: (i, 0, 0)),
            pl.BlockSpec((RB, C, L), lambda i: (i, 0, 0)),
            pl.BlockSpec((RB, C, L), lambda i: (i, 0, 0)),
            pl.BlockSpec((2 * C * L, J), lambda i: (0, 0)),
            pl.BlockSpec((J,), lambda i: (0,)),
        ],
        out_specs=pl.BlockSpec((RB, J), lambda i: (i, 0)),
        out_shape=jax.ShapeDtypeStruct((B, J), jnp.float32),
    )(mag, re, im, gmi, b.astype(jnp.float32))
    return out


# speedup vs baseline: 1.1705x; 1.0721x over previous
import numpy as np

import jax
import jax.numpy as jnp
from jax.experimental import pallas as pl

L = 8192
N1, N2 = 64, 128
C = 2
J = 24
K = 50
RB = 64

_P = jax.lax.Precision.HIGHEST


def _f32(a):
    return jnp.asarray(a, jnp.float32)


_a1 = np.arange(N1)
_a2 = np.arange(N2)
_W1 = np.exp(-2j * np.pi * np.outer(_a1, _a1) / N1)
_W2 = np.exp(-2j * np.pi * np.outer(_a2, _a2) / N2)
_TW = np.exp(-2j * np.pi * np.outer(_a2, _a1) / L)


def _gmat_kernel(wt_ref, w1r_ref, w1i_ref, w2r_ref, w2i_ref, twr_ref, twi_ref,
                 gm_ref):
    w1r, w1i = w1r_ref[...], w1i_ref[...]
    w2r, w2i = w2r_ref[...], w2i_ref[...]
    twr, twi = twr_ref[...], twi_ref[...]

    xm = wt_ref[...].reshape(C * J, N1, N2)
    dn = (((1,), (0,)), ((), ()))
    yr = jax.lax.dot_general(xm, w1r, dn, precision=_P,
                             preferred_element_type=jnp.float32)
    yi = jax.lax.dot_general(xm, w1i, dn, precision=_P,
                             preferred_element_type=jnp.float32)
    zr = yr * twr[None] - yi * twi[None]
    zi = yr * twi[None] + yi * twr[None]
    fr = (jax.lax.dot_general(zr, w2r, dn, precision=_P,
                              preferred_element_type=jnp.float32)
          - jax.lax.dot_general(zi, w2i, dn, precision=_P,
                                preferred_element_type=jnp.float32))
    fi = (jax.lax.dot_general(zr, w2i, dn, precision=_P,
                              preferred_element_type=jnp.float32)
          + jax.lax.dot_general(zi, w2r, dn, precision=_P,
                                preferred_element_type=jnp.float32))
    fr = jnp.swapaxes(fr, 1, 2).reshape(C, J, L)
    fi = jnp.swapaxes(fi, 1, 2).reshape(C, J, L)
    scale = jnp.float32(1.0 / L)
    for c in range(C):
        gm_ref[c, 0, :, :] = jnp.swapaxes(fr[c], 0, 1) * scale
        gm_ref[c, 1, :, :] = jnp.swapaxes(fi[c], 0, 1) * scale


def _main_kernel(mag_ref, re_ref, im_ref, gm_ref, b_ref, out_ref):
    mag = mag_ref[...]
    bits = jax.lax.bitcast_convert_type(mag, jnp.int32)
    cur = jnp.zeros((RB, C, 1), jnp.int32)
    for bit in range(30, -1, -1):
        cand = cur | jnp.int32(1 << bit)
        cnt = jnp.sum((bits >= cand).astype(jnp.int32), axis=-1, keepdims=True)
        cur = jnp.where(cnt >= K, cand, cur)
    kth = jax.lax.bitcast_convert_type(cur, jnp.float32)
    mask = (mag >= kth).astype(jnp.float32)

    are = re_ref[...] * mask
    aim = im_ref[...] * mask
    a = jnp.concatenate(
        [are[:, 0], aim[:, 0], are[:, 1], aim[:, 1]], axis=-1)
    gm = gm_ref[...]
    dn = (((1,), (0,)), ((), ()))
    out = jax.lax.dot_general(a, gm, dn, precision=_P,
                              preferred_element_type=jnp.float32)
    out_ref[...] = out + b_ref[...][None, :]


def kernel(x01, W, b):
    B = x01.shape[0]
    x = 2.0 * x01.astype(jnp.float32) - 1.0
    X = jnp.fft.fft(x, axis=-1)
    mag = jnp.abs(X)
    re = jnp.real(X)
    im = jnp.imag(X)

    wt = W.astype(jnp.float32).reshape(C, L, J).transpose(0, 2, 1).reshape(C * J, L)
    gm = pl.pallas_call(
        _gmat_kernel,
        out_shape=jax.ShapeDtypeStruct((C, 2, L, J), jnp.float32),
    )(wt, _f32(_W1.real), _f32(_W1.imag), _f32(_W2.real), _f32(_W2.imag),
      _f32(_TW.real), _f32(_TW.imag))

    gmi = gm.reshape(2 * C * L, J)

    grid = (B // RB,)
    out = pl.pallas_call(
        _main_kernel,
        grid=grid,
        in_specs=[
            pl.BlockSpec((RB, C, L), lambda i: (i, 0, 0)),
            pl.BlockSpec((RB, C, L), lambda i: (i, 0, 0)),
            pl.BlockSpec((RB, C, L), lambda i: (i, 0, 0)),
            pl.BlockSpec((2 * C * L, J), lambda i: (0, 0)),
            pl.BlockSpec((J,), lambda i: (0,)),
        ],
        out_specs=pl.BlockSpec((RB, J), lambda i: (i, 0)),
        out_shape=jax.ShapeDtypeStruct((B, J), jnp.float32),
    )(mag, re, im, gmi, b.astype(jnp.float32))
    return out
